# baseline (speedup 1.0000x reference)
import functools

import jax
import jax.numpy as jnp
from jax import lax
from jax.experimental import pallas as pl
from jax.experimental.pallas import tpu as pltpu
from jax.experimental.pallas import tpu_sc as plsc

N_NODES = 10000
N_PAD = 10240
H = 128
N_EDGES = 320000
EDGE_BATCH = 64
NB = 320
CH = 64
NCH = NB // CH
EPT = NB * EDGE_BATCH
E_PAD = 16 * EPT
ROWS_PER_TILE = N_PAD // 16


def _proj_body(x_ref, w_ref, b_ref, o_ref):
    o_ref[0] = (
        jnp.dot(x_ref[...], w_ref[0], preferred_element_type=jnp.float32)
        + b_ref[0]
    )


def _project(x, wt, b):
    return pl.pallas_call(
        _proj_body,
        grid=(4,),
        in_specs=[
            pl.BlockSpec((N_NODES, H), lambda k: (0, 0)),
            pl.BlockSpec((1, H, H), lambda k: (k, 0, 0)),
            pl.BlockSpec((1, 1, H), lambda k: (k, 0, 0)),
        ],
        out_specs=pl.BlockSpec((1, N_NODES, H), lambda k: (k, 0, 0)),
        out_shape=jax.ShapeDtypeStruct((4, N_NODES, H), jnp.float32),
    )(x, wt, b)


_MESH = plsc.VectorSubcoreMesh(core_axis_name="c", subcore_axis_name="s")


@functools.partial(
    pl.kernel,
    mesh=_MESH,
    out_type=jax.ShapeDtypeStruct((4, N_PAD, H), jnp.float32),
    scratch_types=[
        pltpu.VMEM((CH, EDGE_BATCH), jnp.int32),
        pltpu.VMEM((CH, EDGE_BATCH), jnp.int32),
        pltpu.VMEM((EDGE_BATCH, H), jnp.float32),
        pltpu.VMEM((EDGE_BATCH, H), jnp.float32),
        pltpu.VMEM((EDGE_BATCH, H), jnp.float32),
        pltpu.VMEM((EDGE_BATCH, H), jnp.float32),
        pltpu.VMEM_SHARED((N_PAD, H), jnp.float32),
        pltpu.SemaphoreType.DMA,
        pltpu.SemaphoreType.DMA,
        pltpu.SemaphoreType.DMA,
        pltpu.SemaphoreType.DMA,
        pltpu.SemaphoreType.DMA,
        pltpu.SemaphoreType.DMA,
        pltpu.SemaphoreType.DMA,
        pltpu.SemaphoreType.DMA,
    ],
)
def _aggregate(t0, t1, t2, t3, zeros_hbm, src_hbm, dst_hbm, out,
               src_v, dst_v, rows0, rows1, rows2, rows3, acc,
               sg0, sg1, sg2, sg3, ss0, ss1, ss2, ss3):
    core = lax.axis_index("c")
    tile = lax.axis_index("s")
    tabs = (t0, t1, t2, t3)
    rows = (rows0, rows1, rows2, rows3)
    sg = (sg0, sg1, sg2, sg3)
    ss = (ss0, ss1, ss2, ss3)

    def run_chunk(tab):
        def outer(g, carry):
            idx_rows = pl.ds(g * CH, CH)
            pltpu.sync_copy(src_hbm.at[tile].at[idx_rows], src_v)
            pltpu.sync_copy(dst_hbm.at[tile].at[idx_rows], dst_v)
            for b in range(2):
                pltpu.async_copy(tab.at[src_v.at[b]], rows[b], sg[b])

            def quad(p, c2):
                for b in range(4):
                    j = 4 * p + b
                    ob = (b + 2) % 4
                    pltpu.make_async_copy(
                        tab.at[src_v.at[j]], rows[b], sg[b]).wait()
                    pltpu.async_copy(
                        rows[b], acc.at[dst_v.at[j]], ss[b], add=True)
                    if b >= 2:
                        pltpu.make_async_copy(
                            rows[ob], acc.at[dst_v.at[j - 2]], ss[ob]).wait()

                        @pl.when(j + 2 < CH)
                        def _():
                            pltpu.async_copy(
                                tab.at[src_v.at[j + 2]], rows[ob], sg[ob])
                    else:
                        @pl.when(j >= 2)
                        def _():
                            pltpu.make_async_copy(
                                rows[ob], acc.at[dst_v.at[j - 2]],
                                ss[ob]).wait()
                        pltpu.async_copy(
                            tab.at[src_v.at[j + 2]], rows[ob], sg[ob])
                return c2
            lax.fori_loop(0, CH // 4, quad, 0)
            pltpu.make_async_copy(
                rows[2], acc.at[dst_v.at[CH - 2]], ss[2]).wait()
            pltpu.make_async_copy(
                rows[3], acc.at[dst_v.at[CH - 1]], ss[3]).wait()
            return carry
        lax.fori_loop(0, NCH, outer, 0)

    my_rows = pl.ds(tile * ROWS_PER_TILE, ROWS_PER_TILE)
    for ci in range(2):
        pltpu.sync_copy(zeros_hbm, acc.at[my_rows])
        plsc.subcore_barrier()

        @pl.when(core == 0)
        def _():
            run_chunk(tabs[ci])

        @pl.when(core == 1)
        def _():
            run_chunk(tabs[2 + ci])

        plsc.subcore_barrier()

        @pl.when(core == 0)
        def _():
            pltpu.sync_copy(acc.at[my_rows], out.at[ci].at[my_rows])

        @pl.when(core == 1)
        def _():
            pltpu.sync_copy(acc.at[my_rows], out.at[2 + ci].at[my_rows])

        plsc.subcore_barrier()


_GR = 400


def _gates_body(p_ref, s_ref, h_ref, c_ref):
    i = jax.nn.sigmoid(p_ref[0] + s_ref[0])
    o = jax.nn.sigmoid(p_ref[1] + s_ref[1])
    u = jnp.tanh(p_ref[2] + s_ref[2])
    c_new = i * u + jax.nn.sigmoid(p_ref[3]) * s_ref[3]
    h_ref[...] = o * jnp.tanh(c_new)
    c_ref[...] = c_new


def _gates(proj, s):
    return pl.pallas_call(
        _gates_body,
        grid=(N_NODES // _GR,),
        in_specs=[
            pl.BlockSpec((4, _GR, H), lambda r: (0, r, 0)),
            pl.BlockSpec((4, _GR, H), lambda r: (0, r, 0)),
        ],
        out_specs=[
            pl.BlockSpec((_GR, H), lambda r: (r, 0)),
            pl.BlockSpec((_GR, H), lambda r: (r, 0)),
        ],
        out_shape=[
            jax.ShapeDtypeStruct((N_NODES, H), jnp.float32),
            jax.ShapeDtypeStruct((N_NODES, H), jnp.float32),
        ],
    )(proj, s)


def kernel(x, c, edge_index, W_iou, b_iou, W_f, b_f):
    wt = jnp.stack([
        W_iou[0:H].T, W_iou[H:2 * H].T, W_iou[2 * H:3 * H].T, W_f.T,
    ])
    b = jnp.stack([
        b_iou[:, 0:H], b_iou[:, H:2 * H], b_iou[:, 2 * H:3 * H], b_f,
    ])

    ei = edge_index.astype(jnp.int32)
    pad = E_PAD - N_EDGES
    src_p = jnp.concatenate([ei[0], jnp.zeros((pad,), jnp.int32)])
    dst_p = jnp.concatenate(
        [ei[1], N_NODES + (jnp.arange(pad, dtype=jnp.int32) % (N_PAD - N_NODES))]
    )
    src_r = src_p.reshape(16, NB, EDGE_BATCH)
    dst_r = dst_p.reshape(16, NB, EDGE_BATCH)
    zeros_hbm = jnp.zeros((ROWS_PER_TILE, H), jnp.float32)

    proj = _project(x, wt, b)
    s = _aggregate(proj[0], proj[1], proj[2], c, zeros_hbm, src_r, dst_r)
    h, c_new = _gates(proj, s)
    return (h, c_new)


# speedup vs baseline: 1.0584x; 1.0584x over previous
import functools

import jax
import jax.numpy as jnp
from jax import lax
from jax.experimental import pallas as pl
from jax.experimental.pallas import tpu as pltpu
from jax.experimental.pallas import tpu_sc as plsc

N_NODES = 10000
N_PAD = 10112
H = 128
N_EDGES = 320000
EDGE_BATCH = 128
NB = 160
CH = 32
NCH = NB // CH
EPT = NB * EDGE_BATCH
E_PAD = 16 * EPT
ROWS_PER_TILE = N_PAD // 16


def _proj_body(x_ref, w_ref, b_ref, o_ref):
    o_ref[0] = (
        jnp.dot(x_ref[...], w_ref[0], preferred_element_type=jnp.float32)
        + b_ref[0]
    )


def _project(x, wt, b):
    return pl.pallas_call(
        _proj_body,
        grid=(4,),
        in_specs=[
            pl.BlockSpec((N_NODES, H), lambda k: (0, 0)),
            pl.BlockSpec((1, H, H), lambda k: (k, 0, 0)),
            pl.BlockSpec((1, 1, H), lambda k: (k, 0, 0)),
        ],
        out_specs=pl.BlockSpec((1, N_NODES, H), lambda k: (k, 0, 0)),
        out_shape=jax.ShapeDtypeStruct((4, N_NODES, H), jnp.float32),
    )(x, wt, b)


_MESH = plsc.VectorSubcoreMesh(core_axis_name="c", subcore_axis_name="s")


@functools.partial(
    pl.kernel,
    mesh=_MESH,
    out_type=jax.ShapeDtypeStruct((4, N_PAD, H), jnp.float32),
    scratch_types=[
        pltpu.VMEM((CH, EDGE_BATCH), jnp.int32),
        pltpu.VMEM((CH, EDGE_BATCH), jnp.int32),
        pltpu.VMEM((CH, EDGE_BATCH), jnp.int32),
        pltpu.VMEM((CH, EDGE_BATCH), jnp.int32),
        pltpu.VMEM((EDGE_BATCH, H), jnp.float32),
        pltpu.VMEM((EDGE_BATCH, H), jnp.float32),
        pltpu.VMEM_SHARED((N_PAD, H), jnp.float32),
        pltpu.SemaphoreType.DMA,
        pltpu.SemaphoreType.DMA,
        pltpu.SemaphoreType.DMA,
        pltpu.SemaphoreType.DMA,
    ],
)
def _aggregate(t0, t1, t2, t3, zeros_hbm, src_hbm, dst_hbm, out,
               sv0, dv0, sv1, dv1, rows0, rows1, acc,
               sg0, sg1, si0, si1):
    core = lax.axis_index("c")
    tile = lax.axis_index("s")
    tabs = (t0, t1, t2, t3)
    rows = (rows0, rows1)
    sg = (sg0, sg1)
    svs = (sv0, sv1)
    dvs = (dv0, dv1)
    si = (si0, si1)

    def stage(g, sync):
        blk = pl.ds(g * CH, CH)
        if sync:
            pltpu.sync_copy(src_hbm.at[tile].at[blk], svs[g % 2])
            pltpu.sync_copy(dst_hbm.at[tile].at[blk], dvs[g % 2])
        else:
            pltpu.async_copy(src_hbm.at[tile].at[blk], svs[g % 2], si[g % 2])
            pltpu.async_copy(dst_hbm.at[tile].at[blk], dvs[g % 2], si[g % 2])

    def stage_wait(g):
        blk = pl.ds(g * CH, CH)
        pltpu.make_async_copy(
            src_hbm.at[tile].at[blk], svs[g % 2], si[g % 2]).wait()
        pltpu.make_async_copy(
            dst_hbm.at[tile].at[blk], dvs[g % 2], si[g % 2]).wait()

    def run_chunk(tab):
        stage(0, True)
        stage(1, False)
        for b in range(2):
            pltpu.async_copy(tab.at[svs[0].at[b]], rows[b], sg[b])

        for g in range(NCH):
            sv, dv = svs[g % 2], dvs[g % 2]
            nsv = svs[(g + 1) % 2]

            def body(p, c2):
                for b in range(2):
                    j = 2 * p + b
                    pltpu.make_async_copy(
                        tab.at[sv.at[j]], rows[b], sg[b]).wait()
                    pltpu.sync_copy(rows[b], acc.at[dv.at[j]], add=True)
                    pltpu.async_copy(tab.at[sv.at[j + 2]], rows[b], sg[b])
                return c2
            lax.fori_loop(0, CH // 2 - 1, body, 0)

            if g + 1 < NCH:
                stage_wait(g + 1)
            for b in range(2):
                j = CH - 2 + b
                pltpu.make_async_copy(
                    tab.at[sv.at[j]], rows[b], sg[b]).wait()
                pltpu.sync_copy(rows[b], acc.at[dv.at[j]], add=True)
                if g + 1 < NCH:
                    pltpu.async_copy(tab.at[nsv.at[b]], rows[b], sg[b])
            if g + 2 < NCH:
                stage(g + 2, False)

    my_rows = pl.ds(tile * ROWS_PER_TILE, ROWS_PER_TILE)
    for ci in range(2):
        pltpu.sync_copy(zeros_hbm, acc.at[my_rows])
        plsc.subcore_barrier()

        @pl.when(core == 0)
        def _():
            run_chunk(tabs[ci])

        @pl.when(core == 1)
        def _():
            run_chunk(tabs[2 + ci])

        plsc.subcore_barrier()

        @pl.when(core == 0)
        def _():
            pltpu.sync_copy(acc.at[my_rows], out.at[ci].at[my_rows])

        @pl.when(core == 1)
        def _():
            pltpu.sync_copy(acc.at[my_rows], out.at[2 + ci].at[my_rows])

        plsc.subcore_barrier()


_GR = 400


def _gates_body(p_ref, s_ref, h_ref, c_ref):
    i = jax.nn.sigmoid(p_ref[0] + s_ref[0])
    o = jax.nn.sigmoid(p_ref[1] + s_ref[1])
    u = jnp.tanh(p_ref[2] + s_ref[2])
    c_new = i * u + jax.nn.sigmoid(p_ref[3]) * s_ref[3]
    h_ref[...] = o * jnp.tanh(c_new)
    c_ref[...] = c_new


def _gates(proj, s):
    return pl.pallas_call(
        _gates_body,
        grid=(N_NODES // _GR,),
        in_specs=[
            pl.BlockSpec((4, _GR, H), lambda r: (0, r, 0)),
            pl.BlockSpec((4, _GR, H), lambda r: (0, r, 0)),
        ],
        out_specs=[
            pl.BlockSpec((_GR, H), lambda r: (r, 0)),
            pl.BlockSpec((_GR, H), lambda r: (r, 0)),
        ],
        out_shape=[
            jax.ShapeDtypeStruct((N_NODES, H), jnp.float32),
            jax.ShapeDtypeStruct((N_NODES, H), jnp.float32),
        ],
    )(proj, s)


def kernel(x, c, edge_index, W_iou, b_iou, W_f, b_f):
    wt = jnp.stack([
        W_iou[0:H].T, W_iou[H:2 * H].T, W_iou[2 * H:3 * H].T, W_f.T,
    ])
    b = jnp.stack([
        b_iou[:, 0:H], b_iou[:, H:2 * H], b_iou[:, 2 * H:3 * H], b_f,
    ])

    ei = edge_index.astype(jnp.int32)
    pad = E_PAD - N_EDGES
    src_p = jnp.concatenate([ei[0], jnp.zeros((pad,), jnp.int32)])
    dst_p = jnp.concatenate(
        [ei[1], N_NODES + (jnp.arange(pad, dtype=jnp.int32) % (N_PAD - N_NODES))]
    )
    src_r = src_p.reshape(16, NB, EDGE_BATCH)
    dst_r = dst_p.reshape(16, NB, EDGE_BATCH)
    zeros_hbm = jnp.zeros((ROWS_PER_TILE, H), jnp.float32)

    proj = _project(x, wt, b)
    s = _aggregate(proj[0], proj[1], proj[2], c, zeros_hbm, src_r, dst_r)
    h, c_new = _gates(proj, s)
    return (h, c_new)


# speedup vs baseline: 1.6454x; 1.5546x over previous
import functools

import jax
import jax.numpy as jnp
from jax import lax
from jax.experimental import pallas as pl
from jax.experimental.pallas import tpu as pltpu
from jax.experimental.pallas import tpu_sc as plsc

N_NODES = 10000
N_PAD = 10112
H = 128
N_EDGES = 320000
EDGE_BATCH = 128
NB = 158
CH = 32
BLOCKS = ((0, 32), (32, 32), (64, 32), (96, 32), (128, 30))
EPT = NB * EDGE_BATCH
E_PAD = 16 * EPT
ROWS_PER_TILE = N_PAD // 16


def _proj_body(x_ref, w_ref, b_ref, o_ref):
    o_ref[0] = (
        jnp.dot(x_ref[...], w_ref[0], preferred_element_type=jnp.float32)
        + b_ref[0]
    )


def _project(x, wt, b):
    return pl.pallas_call(
        _proj_body,
        grid=(4,),
        in_specs=[
            pl.BlockSpec((N_NODES, H), lambda k: (0, 0)),
            pl.BlockSpec((1, H, H), lambda k: (k, 0, 0)),
            pl.BlockSpec((1, 1, H), lambda k: (k, 0, 0)),
        ],
        out_specs=pl.BlockSpec((1, N_NODES, H), lambda k: (k, 0, 0)),
        out_shape=jax.ShapeDtypeStruct((4, N_NODES, H), jnp.float32),
    )(x, wt, b)


_MESH = plsc.VectorSubcoreMesh(core_axis_name="c", subcore_axis_name="s")


@functools.partial(
    pl.kernel,
    mesh=_MESH,
    out_type=jax.ShapeDtypeStruct((4, N_PAD, H), jnp.float32),
    scratch_types=[
        pltpu.VMEM((CH, EDGE_BATCH), jnp.int32),
        pltpu.VMEM((CH, EDGE_BATCH), jnp.int32),
        pltpu.VMEM((CH, EDGE_BATCH), jnp.int32),
        pltpu.VMEM((CH, EDGE_BATCH), jnp.int32),
        pltpu.VMEM((EDGE_BATCH, H), jnp.float32),
        pltpu.VMEM((EDGE_BATCH, H), jnp.float32),
        pltpu.VMEM_SHARED((N_PAD, H), jnp.float32),
        pltpu.SemaphoreType.DMA,
        pltpu.SemaphoreType.DMA,
        pltpu.SemaphoreType.DMA,
        pltpu.SemaphoreType.DMA,
    ],
)
def _aggregate(t0, t1, t2, t3, zeros_hbm, src_hbm, dst_hbm, out,
               sv0, dv0, sv1, dv1, rows0, rows1, acc,
               sg0, sg1, si0, si1):
    core = lax.axis_index("c")
    tile = lax.axis_index("s")
    tabs = (t0, t1, t2, t3)
    rows = (rows0, rows1)
    sg = (sg0, sg1)
    svs = (sv0, sv1)
    dvs = (dv0, dv1)
    si = (si0, si1)

    def stage(g, sync):
        off, size = BLOCKS[g]
        blk = pl.ds(off, size)
        dst_blk = pl.ds(0, size)
        if sync:
            pltpu.sync_copy(src_hbm.at[tile].at[blk], svs[g % 2].at[dst_blk])
            pltpu.sync_copy(dst_hbm.at[tile].at[blk], dvs[g % 2].at[dst_blk])
        else:
            pltpu.async_copy(
                src_hbm.at[tile].at[blk], svs[g % 2].at[dst_blk], si[g % 2])
            pltpu.async_copy(
                dst_hbm.at[tile].at[blk], dvs[g % 2].at[dst_blk], si[g % 2])

    def stage_wait(g):
        off, size = BLOCKS[g]
        blk = pl.ds(off, size)
        dst_blk = pl.ds(0, size)
        pltpu.make_async_copy(
            src_hbm.at[tile].at[blk], svs[g % 2].at[dst_blk],
            si[g % 2]).wait()
        pltpu.make_async_copy(
            dst_hbm.at[tile].at[blk], dvs[g % 2].at[dst_blk],
            si[g % 2]).wait()

    def run_chunk(tab):
        stage(0, True)
        stage(1, False)
        for b in range(2):
            pltpu.async_copy(tab.at[svs[0].at[b]], rows[b], sg[b])

        ngb = len(BLOCKS)
        for g in range(ngb):
            size = BLOCKS[g][1]
            sv, dv = svs[g % 2], dvs[g % 2]
            nsv = svs[(g + 1) % 2]

            def body(p, c2):
                for b in range(2):
                    j = 2 * p + b
                    pltpu.make_async_copy(
                        tab.at[sv.at[j]], rows[b], sg[b]).wait()
                    pltpu.sync_copy(rows[b], acc.at[dv.at[j]], add=True)
                    pltpu.async_copy(tab.at[sv.at[j + 2]], rows[b], sg[b])
                return c2
            lax.fori_loop(0, size // 2 - 1, body, 0)

            if g + 1 < ngb:
                stage_wait(g + 1)
            for b in range(2):
                j = size - 2 + b
                pltpu.make_async_copy(
                    tab.at[sv.at[j]], rows[b], sg[b]).wait()
                pltpu.sync_copy(rows[b], acc.at[dv.at[j]], add=True)
                if g + 1 < ngb:
                    pltpu.async_copy(tab.at[nsv.at[b]], rows[b], sg[b])
            if g + 2 < ngb:
                stage(g + 2, False)

    my_rows = pl.ds(tile * ROWS_PER_TILE, ROWS_PER_TILE)
    for ci in range(2):
        pltpu.sync_copy(zeros_hbm, acc.at[my_rows])
        plsc.subcore_barrier()

        @pl.when(core == 0)
        def _():
            run_chunk(tabs[ci])

        @pl.when(core == 1)
        def _():
            run_chunk(tabs[2 + ci])

        plsc.subcore_barrier()

        @pl.when(core == 0)
        def _():
            pltpu.sync_copy(acc.at[my_rows], out.at[ci].at[my_rows])

        @pl.when(core == 1)
        def _():
            pltpu.sync_copy(acc.at[my_rows], out.at[2 + ci].at[my_rows])

        plsc.subcore_barrier()


_GR = 400


def _gates_body(p_ref, s_ref, h_ref, c_ref):
    i = jax.nn.sigmoid(p_ref[0] + s_ref[0])
    o = jax.nn.sigmoid(p_ref[1] + s_ref[1])
    u = jnp.tanh(p_ref[2] + s_ref[2])
    c_new = i * u + jax.nn.sigmoid(p_ref[3]) * s_ref[3]
    h_ref[...] = o * jnp.tanh(c_new)
    c_ref[...] = c_new


def _gates(proj, s):
    return pl.pallas_call(
        _gates_body,
        grid=(N_NODES // _GR,),
        in_specs=[
            pl.BlockSpec((4, _GR, H), lambda r: (0, r, 0)),
            pl.BlockSpec((4, _GR, H), lambda r: (0, r, 0)),
        ],
        out_specs=[
            pl.BlockSpec((_GR, H), lambda r: (r, 0)),
            pl.BlockSpec((_GR, H), lambda r: (r, 0)),
        ],
        out_shape=[
            jax.ShapeDtypeStruct((N_NODES, H), jnp.float32),
            jax.ShapeDtypeStruct((N_NODES, H), jnp.float32),
        ],
    )(proj, s)


def kernel(x, c, edge_index, W_iou, b_iou, W_f, b_f):
    wt = jnp.stack([
        W_iou[0:H].T, W_iou[H:2 * H].T, W_iou[2 * H:3 * H].T, W_f.T,
    ])
    b = jnp.stack([
        b_iou[:, 0:H], b_iou[:, H:2 * H], b_iou[:, 2 * H:3 * H], b_f,
    ])

    ei = edge_index.astype(jnp.int32)
    pad = E_PAD - N_EDGES
    src_p = jnp.concatenate([ei[0], jnp.zeros((pad,), jnp.int32)])
    dst_p = jnp.concatenate(
        [ei[1], N_NODES + (jnp.arange(pad, dtype=jnp.int32) % (N_PAD - N_NODES))]
    )
    src_r = src_p.reshape(16, NB, EDGE_BATCH)
    dst_r = dst_p.reshape(16, NB, EDGE_BATCH)
    zeros_hbm = jnp.zeros((ROWS_PER_TILE, H), jnp.float32)

    proj = _project(x, wt, b)
    s = _aggregate(proj[0], proj[1], proj[2], c, zeros_hbm, src_r, dst_r)
    h, c_new = _gates(proj, s)
    return (h, c_new)


# speedup vs baseline: 2.7796x; 1.6893x over previous
import functools

import jax
import jax.numpy as jnp
from jax import lax
from jax.experimental import pallas as pl
from jax.experimental.pallas import tpu as pltpu
from jax.experimental.pallas import tpu_sc as plsc

N_NODES = 10000
N_PAD = 10112
H = 128
N_EDGES = 320000
EDGE_BATCH = 128
NB = 158
CH = 32
BLOCKS = ((0, 32), (32, 32), (64, 32), (96, 32), (128, 30))
EPT = NB * EDGE_BATCH
E_PAD = 16 * EPT
ROWS_PER_TILE = N_PAD // 16


def _proj_body(x_ref, w_ref, b_ref, o_ref):
    o_ref[0] = (
        jnp.dot(x_ref[...], w_ref[0], preferred_element_type=jnp.float32)
        + b_ref[0]
    )


def _project(x, wt, b):
    return pl.pallas_call(
        _proj_body,
        grid=(4,),
        in_specs=[
            pl.BlockSpec((N_NODES, H), lambda k: (0, 0)),
            pl.BlockSpec((1, H, H), lambda k: (k, 0, 0)),
            pl.BlockSpec((1, 1, H), lambda k: (k, 0, 0)),
        ],
        out_specs=pl.BlockSpec((1, N_NODES, H), lambda k: (k, 0, 0)),
        out_shape=jax.ShapeDtypeStruct((4, N_NODES, H), jnp.float32),
    )(x, wt, b)


_MESH = plsc.VectorSubcoreMesh(core_axis_name="c", subcore_axis_name="s")


@functools.partial(
    pl.kernel,
    mesh=_MESH,
    out_type=jax.ShapeDtypeStruct((4, N_PAD, H), jnp.float32),
    scratch_types=[
        pltpu.VMEM((CH, EDGE_BATCH), jnp.int32),
        pltpu.VMEM((CH, EDGE_BATCH), jnp.int32),
        pltpu.VMEM((CH, EDGE_BATCH), jnp.int32),
        pltpu.VMEM((CH, EDGE_BATCH), jnp.int32),
        pltpu.VMEM((EDGE_BATCH, H), jnp.float32),
        pltpu.VMEM((EDGE_BATCH, H), jnp.float32),
        pltpu.VMEM_SHARED((N_PAD, H), jnp.float32),
        pltpu.SemaphoreType.DMA,
        pltpu.SemaphoreType.DMA,
        pltpu.SemaphoreType.DMA,
        pltpu.SemaphoreType.DMA,
    ],
)
def _aggregate(t0, t1, t2, t3, zeros_hbm, src_hbm, dst_hbm, out,
               sv0, dv0, sv1, dv1, rows0, rows1, acc,
               sg0, sg1, si0, si1):
    core = lax.axis_index("c")
    tile = lax.axis_index("s")
    tabs = (t0, t1, t2, t3)
    rows = (rows0, rows1)
    sg = (sg0, sg1)
    svs = (sv0, sv1)
    dvs = (dv0, dv1)
    si = (si0, si1)

    def stage(g, sync):
        off, size = BLOCKS[g]
        blk = pl.ds(off, size)
        dst_blk = pl.ds(0, size)
        if sync:
            pltpu.sync_copy(src_hbm.at[tile].at[blk], svs[g % 2].at[dst_blk])
            pltpu.sync_copy(dst_hbm.at[tile].at[blk], dvs[g % 2].at[dst_blk])
        else:
            pltpu.async_copy(
                src_hbm.at[tile].at[blk], svs[g % 2].at[dst_blk], si[g % 2])
            pltpu.async_copy(
                dst_hbm.at[tile].at[blk], dvs[g % 2].at[dst_blk], si[g % 2])

    def stage_wait(g):
        off, size = BLOCKS[g]
        blk = pl.ds(off, size)
        dst_blk = pl.ds(0, size)
        pltpu.make_async_copy(
            src_hbm.at[tile].at[blk], svs[g % 2].at[dst_blk],
            si[g % 2]).wait()
        pltpu.make_async_copy(
            dst_hbm.at[tile].at[blk], dvs[g % 2].at[dst_blk],
            si[g % 2]).wait()

    def run_chunk(tab):
        stage(0, True)
        stage(1, False)
        for b in range(2):
            pltpu.async_copy(tab.at[svs[0].at[b]], rows[b], sg[b])

        ngb = len(BLOCKS)
        for g in range(ngb):
            size = BLOCKS[g][1]
            sv, dv = svs[g % 2], dvs[g % 2]
            nsv = svs[(g + 1) % 2]

            def body(p, c2):
                for b in range(2):
                    j = 2 * p + b
                    pltpu.make_async_copy(
                        tab.at[sv.at[j]], rows[b], sg[b]).wait()
                    pltpu.sync_copy(rows[b], acc.at[dv.at[j]], add=True)
                    pltpu.async_copy(tab.at[sv.at[j + 2]], rows[b], sg[b])
                return c2
            lax.fori_loop(0, size // 2 - 1, body, 0)

            if g + 1 < ngb:
                stage_wait(g + 1)
            for b in range(2):
                j = size - 2 + b
                pltpu.make_async_copy(
                    tab.at[sv.at[j]], rows[b], sg[b]).wait()
                pltpu.sync_copy(rows[b], acc.at[dv.at[j]], add=True)
                if g + 1 < ngb:
                    pltpu.async_copy(tab.at[nsv.at[b]], rows[b], sg[b])
            if g + 2 < ngb:
                stage(g + 2, False)

    my_rows = pl.ds(tile * ROWS_PER_TILE, ROWS_PER_TILE)
    for ci in range(2):
        pltpu.sync_copy(zeros_hbm, acc.at[my_rows])
        plsc.subcore_barrier()

        @pl.when(core == 0)
        def _():
            run_chunk(tabs[ci])

        @pl.when(core == 1)
        def _():
            run_chunk(tabs[2 + ci])

        plsc.subcore_barrier()

        @pl.when(core == 0)
        def _():
            pltpu.sync_copy(acc.at[my_rows], out.at[ci].at[my_rows])

        @pl.when(core == 1)
        def _():
            pltpu.sync_copy(acc.at[my_rows], out.at[2 + ci].at[my_rows])

        plsc.subcore_barrier()


_GR = 400


def _gates_body(p_ref, s_ref, h_ref, c_ref):
    i = jax.nn.sigmoid(p_ref[0] + s_ref[0])
    o = jax.nn.sigmoid(p_ref[1] + s_ref[1])
    u = jnp.tanh(p_ref[2] + s_ref[2])
    c_new = i * u + jax.nn.sigmoid(p_ref[3]) * s_ref[3]
    h_ref[...] = o * jnp.tanh(c_new)
    c_ref[...] = c_new


def _gates(proj, s):
    return pl.pallas_call(
        _gates_body,
        grid=(N_NODES // _GR,),
        in_specs=[
            pl.BlockSpec((4, _GR, H), lambda r: (0, r, 0)),
            pl.BlockSpec((4, _GR, H), lambda r: (0, r, 0)),
        ],
        out_specs=[
            pl.BlockSpec((_GR, H), lambda r: (r, 0)),
            pl.BlockSpec((_GR, H), lambda r: (r, 0)),
        ],
        out_shape=[
            jax.ShapeDtypeStruct((N_NODES, H), jnp.float32),
            jax.ShapeDtypeStruct((N_NODES, H), jnp.float32),
        ],
    )(proj, s)


def kernel(x, c, edge_index, W_iou, b_iou, W_f, b_f):
    wt = jnp.stack([
        W_iou[0:H].T, W_iou[H:2 * H].T, W_iou[2 * H:3 * H].T, W_f.T,
    ])
    b = jnp.stack([
        b_iou[:, 0:H], b_iou[:, H:2 * H], b_iou[:, 2 * H:3 * H], b_f,
    ])

    ei = edge_index.astype(jnp.int32)
    pad = E_PAD - N_EDGES
    src_p = jnp.concatenate(
        [ei[0], jnp.arange(pad, dtype=jnp.int32) % N_NODES])
    dst_p = jnp.concatenate(
        [ei[1], N_NODES + (jnp.arange(pad, dtype=jnp.int32) % (N_PAD - N_NODES))]
    )
    src_r = src_p.reshape(NB, 16, EDGE_BATCH).transpose(1, 0, 2)
    dst_r = dst_p.reshape(NB, 16, EDGE_BATCH).transpose(1, 0, 2)
    zeros_hbm = jnp.zeros((ROWS_PER_TILE, H), jnp.float32)

    proj = _project(x, wt, b)
    s = _aggregate(proj[0], proj[1], proj[2], c, zeros_hbm, src_r, dst_r)
    h, c_new = _gates(proj, s)
    return (h, c_new)
